# baseline (speedup 1.0000x reference)
import functools

import jax
import jax.numpy as jnp
from jax import lax
from jax.experimental import pallas as pl
from jax.experimental.pallas import tpu as pltpu
from jax.experimental.pallas import tpu_sc as plsc

NUM_EMBEDDINGS = 1000000
EMBEDDING_DIM = 64
B_I = 16384
B_J = 50
NW = 32

_mesh = plsc.VectorSubcoreMesh(core_axis_name="c", subcore_axis_name="s")

A_FULL = NUM_EMBEDDINGS // 128
A_PER_W = (A_FULL + NW - 1) // NW
A_NBUF = 3


@functools.partial(
    pl.kernel,
    mesh=_mesh,
    out_type=jax.ShapeDtypeStruct((NUM_EMBEDDINGS // 2, 128), jnp.float32),
    scratch_types=[
        pltpu.VMEM((A_NBUF, 64, 128), jnp.float32),
        pltpu.VMEM((2, 64, 128), jnp.float32),
        pltpu.SemaphoreType.DMA((A_NBUF,)),
        pltpu.SemaphoreType.DMA((2,)),
    ],
    compiler_params=pltpu.CompilerParams(use_tc_tiling_on_sc=True, needs_layout_passes=False),
)
def _transpose_kernel(wt_hbm, w2_hbm, in_v, out_v, isem, osem):
    wid = lax.axis_index("s") * 2 + lax.axis_index("c")
    iota = lax.iota(jnp.int32, 16)

    def widx(k):
        return wid + NW * k

    def start_in(k):
        slot = k % A_NBUF
        r0 = widx(k) * 128
        pltpu.async_copy(
            wt_hbm.at[:, pl.ds(r0, 128)], in_v.at[slot], isem.at[slot]
        )

    def wait_in(k):
        slot = k % A_NBUF
        r0 = widx(k) * 128
        pltpu.make_async_copy(
            wt_hbm.at[:, pl.ds(r0, 128)], in_v.at[slot], isem.at[slot]
        ).wait()

    def start_out(k):
        slot = k % 2
        pltpu.async_copy(
            out_v.at[slot], w2_hbm.at[pl.ds(widx(k) * 64, 64)], osem.at[slot]
        )

    def wait_out(k):
        slot = k % 2
        pltpu.make_async_copy(
            out_v.at[slot], w2_hbm.at[pl.ds(widx(k) * 64, 64)], osem.at[slot]
        ).wait()

    def transpose_window(islot, oslot, ncols):
        def body(j2, carry):
            for p in range(2):
                col = jnp.full((16,), 2 * j2 + p, jnp.int32)
                for t in range(4):
                    rows = iota + 16 * t
                    vals = plsc.load_gather(in_v.at[islot], [rows, col])
                    out_v[oslot, j2, pl.ds(64 * p + 16 * t, 16)] = vals
            return carry

        lax.fori_loop(0, ncols // 2, body, 0)

    def valid(k):
        return widx(k) < A_FULL

    for k in range(A_NBUF):
        @pl.when(valid(k))
        def _(k=k):
            start_in(k)

    def body(k, carry):
        @pl.when(valid(k))
        def _():
            wait_in(k)

            @pl.when(k >= 2)
            def _():
                wait_out(k - 2)

            transpose_window(k % A_NBUF, k % 2, 128)
            start_out(k)

            @pl.when(valid(k + A_NBUF))
            def _():
                start_in(k + A_NBUF)

        return carry

    lax.fori_loop(0, A_PER_W, body, 0)

    for d in (2, 1):
        @pl.when(valid(A_PER_W - d))
        def _(d=d):
            wait_out(A_PER_W - d)

    @pl.when(wid == NW - 1)
    def _():
        for c in range(64):
            pltpu.async_copy(
                wt_hbm.at[c, pl.ds(A_FULL * 128, 64)],
                in_v.at[0, c, pl.ds(0, 64)],
                isem.at[0],
            )
        for c in range(64):
            pltpu.make_async_copy(
                wt_hbm.at[c, pl.ds(A_FULL * 128, 64)],
                in_v.at[0, c, pl.ds(0, 64)],
                isem.at[0],
            ).wait()
        transpose_window(0, 0, 64)
        pltpu.sync_copy(
            out_v.at[0, pl.ds(0, 32), :],
            w2_hbm.at[pl.ds(A_FULL * 64, 32)],
        )


B_WINDOWS = B_J * 128
B_PER_W = B_WINDOWS // NW
B_NBUF = 4


@functools.partial(
    pl.kernel,
    mesh=_mesh,
    out_type=jax.ShapeDtypeStruct((B_I * B_J * EMBEDDING_DIM // 128, 128), jnp.float32),
    scratch_types=[
        pltpu.VMEM((B_NBUF, 128), jnp.int32),
        pltpu.VMEM((B_NBUF, 128), jnp.int32),
        pltpu.VMEM((B_NBUF, 128), jnp.int32),
        pltpu.VMEM((B_NBUF, 128, 128), jnp.float32),
        pltpu.VMEM((2, 64, 128), jnp.float32),
        pltpu.SemaphoreType.DMA((B_NBUF,)),
        pltpu.SemaphoreType.DMA((B_NBUF,)),
        pltpu.SemaphoreType.DMA((2,)),
    ],
    compiler_params=pltpu.CompilerParams(use_tc_tiling_on_sc=True, needs_layout_passes=False),
)
def _gather_kernel(idx3_hbm, w2_hbm, out2_hbm, idx_v, pair_v, par_v, g_v, out_v,
                   xsem, gsem, osem):
    wid = lax.axis_index("s") * 2 + lax.axis_index("c")
    iota = lax.iota(jnp.int32, 16)

    def jit_of(k):
        w = wid + NW * k
        return w // 128, w % 128

    def start_idx(k):
        slot = k % B_NBUF
        j, it = jit_of(k)
        pltpu.async_copy(idx3_hbm.at[j, it], idx_v.at[slot], xsem.at[slot])

    def wait_idx(k):
        slot = k % B_NBUF
        j, it = jit_of(k)
        pltpu.make_async_copy(
            idx3_hbm.at[j, it], idx_v.at[slot], xsem.at[slot]
        ).wait()

    def start_gather(k):
        slot = k % B_NBUF
        pltpu.async_copy(w2_hbm.at[pair_v.at[slot]], g_v.at[slot], gsem.at[slot])

    def wait_gather(k):
        slot = k % B_NBUF
        pltpu.make_async_copy(
            w2_hbm.at[pair_v.at[slot]], g_v.at[slot], gsem.at[slot]
        ).wait()

    def obase(k, c8):
        j, it = jit_of(k)
        return j * 8192 + c8 * 1024 + it * 8

    def start_writes(k):
        oslot = k % 2
        for c8 in range(8):
            pltpu.async_copy(
                out_v.at[oslot, pl.ds(8 * c8, 8), :],
                out2_hbm.at[pl.ds(obase(k, c8), 8)],
                osem.at[oslot],
            )

    def wait_writes(k):
        oslot = k % 2
        for c8 in range(8):
            pltpu.make_async_copy(
                out_v.at[oslot, pl.ds(8 * c8, 8), :],
                out2_hbm.at[pl.ds(obase(k, c8), 8)],
                osem.at[oslot],
            ).wait()

    def prep_indices(k):
        slot = k % B_NBUF
        for t in range(8):
            v = idx_v[slot, pl.ds(16 * t, 16)]
            pair_v[slot, pl.ds(16 * t, 16)] = v >> 1
            par_v[slot, pl.ds(16 * t, 16)] = (v & 1) * 64

    def transpose_block(k):
        slot = k % B_NBUF
        oslot = k % 2

        def body(c, carry):
            for t in range(8):
                rows = iota + 16 * t
                cols = par_v[slot, pl.ds(16 * t, 16)] + c
                vals = plsc.load_gather(g_v.at[slot], [rows, cols])
                out_v[oslot, c, pl.ds(16 * t, 16)] = vals
            return carry

        lax.fori_loop(0, 64, body, 0)

    for k in range(B_NBUF):
        start_idx(k)

    LAG = 2

    def body(k, carry):
        @pl.when(k < B_PER_W)
        def _():
            wait_idx(k)
            prep_indices(k)
            start_gather(k)

            @pl.when(k + B_NBUF < B_PER_W)
            def _():
                start_idx(k + B_NBUF)

        @pl.when(k >= LAG)
        def _():
            m = k - LAG
            wait_gather(m)

            @pl.when(m >= 2)
            def _():
                wait_writes(m - 2)

            transpose_block(m)
            start_writes(m)

        return carry

    lax.fori_loop(0, B_PER_W + LAG, body, 0)
    wait_writes(B_PER_W - 2)
    wait_writes(B_PER_W - 1)


def kernel(input_, weight):
    wt = weight.T
    w2 = _transpose_kernel(wt)
    idx3 = input_.T.reshape(B_J, 128, 128).astype(jnp.int32)
    out2 = _gather_kernel(idx3, w2)
    return (
        out2.reshape(B_J, 8, 128, 8, 128)
        .transpose(2, 4, 0, 1, 3)
        .reshape(B_I, B_J, EMBEDDING_DIM)
    )


# speedup vs baseline: 2.7355x; 2.7355x over previous
import functools

import jax
import jax.numpy as jnp
from jax import lax
from jax.experimental import pallas as pl
from jax.experimental.pallas import tpu as pltpu
from jax.experimental.pallas import tpu_sc as plsc

NUM_EMBEDDINGS = 1000000
EMBEDDING_DIM = 64

B_TOTAL = 16384 * 50
NUM_WORKERS = 32
B_PER_W = B_TOTAL // NUM_WORKERS
CHUNK = 128
NCHUNK = B_PER_W // CHUNK

_mesh = plsc.VectorSubcoreMesh(core_axis_name="c", subcore_axis_name="s")


NBUF = 8


@functools.partial(
    pl.kernel,
    mesh=_mesh,
    out_type=jax.ShapeDtypeStruct((B_TOTAL, EMBEDDING_DIM), jnp.float32),
    scratch_types=[
        pltpu.VMEM((NCHUNK, CHUNK), jnp.int32),
        pltpu.VMEM((NBUF, CHUNK, EMBEDDING_DIM), jnp.float32),
        pltpu.SemaphoreType.DMA((NBUF,)),
        pltpu.SemaphoreType.DMA((NBUF,)),
    ],
    compiler_params=pltpu.CompilerParams(use_tc_tiling_on_sc=False),
)
def _gather_kernel(idx_hbm, table_hbm, out_hbm, idx_v, rows_v, gsem, osem):
    wid = lax.axis_index("s") * 2 + lax.axis_index("c")
    base = wid * B_PER_W

    pltpu.sync_copy(idx_hbm.at[wid], idx_v)

    def start_gather(c):
        slot = c % NBUF
        pltpu.async_copy(table_hbm.at[idx_v.at[c]], rows_v.at[slot], gsem.at[slot])

    def wait_gather(c):
        slot = c % NBUF
        pltpu.make_async_copy(
            table_hbm.at[idx_v.at[c]], rows_v.at[slot], gsem.at[slot]
        ).wait()

    def start_write(c):
        slot = c % NBUF
        pltpu.async_copy(
            rows_v.at[slot], out_hbm.at[pl.ds(base + c * CHUNK, CHUNK)], osem.at[slot]
        )

    def wait_write(c):
        slot = c % NBUF
        pltpu.make_async_copy(
            rows_v.at[slot], out_hbm.at[pl.ds(base + c * CHUNK, CHUNK)], osem.at[slot]
        ).wait()

    for c in range(NBUF):
        start_gather(c)

    def body(j, carry):
        wait_gather(j)

        @pl.when(j > 0)
        def _():
            wait_write(j - 1)

            @pl.when(j + NBUF - 1 < NCHUNK)
            def _():
                start_gather(j + NBUF - 1)

        start_write(j)
        return carry

    lax.fori_loop(0, NCHUNK, body, 0)
    wait_write(NCHUNK - 1)


def kernel(input_, weight):
    idx = input_.reshape(NUM_WORKERS, NCHUNK, CHUNK).astype(jnp.int32)
    out = _gather_kernel(idx, weight)
    return out.reshape(input_.shape + (EMBEDDING_DIM,))
